# baseline (speedup 1.0000x reference)
import functools

import jax
import jax.numpy as jnp
from jax import lax
from jax.experimental import pallas as pl
from jax.experimental.pallas import tpu as pltpu
from jax.experimental.pallas import tpu_sc as plsc

VOCAB = 256
EMBED_DIM = 128
BATCH = 16384

NC = 2
NS = 16
NW = NC * NS
B_PER_W = BATCH // NW
CHUNK = 64
N_CHUNKS = B_PER_W // CHUNK
NBUF = 6


def _build():
    mesh = plsc.VectorSubcoreMesh(
        core_axis_name="c", subcore_axis_name="s", num_cores=NC
    )

    @functools.partial(
        pl.kernel,
        mesh=mesh,
        compiler_params=pltpu.CompilerParams(skip_device_barrier=True),
        out_type=jax.ShapeDtypeStruct((BATCH, EMBED_DIM), jnp.float32),
        scratch_types=[
            pltpu.VMEM((B_PER_W,), jnp.int32),
            pltpu.VMEM((NBUF, CHUNK, EMBED_DIM), jnp.float32),
            pltpu.VMEM_SHARED((VOCAB, EMBED_DIM), jnp.float32),
            pltpu.SemaphoreType.DMA((NBUF,)),
            pltpu.SemaphoreType.DMA((NBUF,)),
            pltpu.SemaphoreType.DMA,
            pltpu.SemaphoreType.DMA,
        ],
    )
    def gather_kernel(
        table_hbm, idx_hbm, out_hbm, idx_v, rows_v, table_sp, gsem, wsem,
        tsem, isem,
    ):
        cid = lax.axis_index("c")
        sid = lax.axis_index("s")
        wid = sid * NC + cid
        base = wid * B_PER_W

        rows_per_tile = VOCAB // NS
        trow = sid * rows_per_tile
        stage = pltpu.async_copy(
            table_hbm.at[pl.ds(trow, rows_per_tile)],
            table_sp.at[pl.ds(trow, rows_per_tile)],
            tsem,
        )
        iload = pltpu.async_copy(idx_hbm.at[pl.ds(base, B_PER_W)], idx_v, isem)
        stage.wait()
        iload.wait()
        plsc.subcore_barrier()

        gathers = [None] * N_CHUNKS
        writes = [None] * N_CHUNKS
        for j in range(N_CHUNKS + 1):
            if j < N_CHUNKS:
                s = j % NBUF
                if j >= NBUF:
                    writes[j - NBUF].wait()
                gathers[j] = pltpu.async_copy(
                    table_sp.at[idx_v.at[pl.ds(j * CHUNK, CHUNK)]],
                    rows_v.at[s],
                    gsem.at[s],
                )
            if j >= 1:
                jj = j - 1
                gathers[jj].wait()
                writes[jj] = pltpu.async_copy(
                    rows_v.at[jj % NBUF],
                    out_hbm.at[pl.ds(base + jj * CHUNK, CHUNK)],
                    wsem.at[jj % NBUF],
                )
        for w in writes[-NBUF:]:
            w.wait()

    return gather_kernel


@functools.cache
def _get_gather():
    return _build()


def kernel(input, table):
    idx = input.reshape(-1).astype(jnp.int32)
    return _get_gather()(table, idx)


# speedup vs baseline: 1.0026x; 1.0026x over previous
import functools

import jax
import jax.numpy as jnp
from jax import lax
from jax.experimental import pallas as pl
from jax.experimental.pallas import tpu as pltpu
from jax.experimental.pallas import tpu_sc as plsc

VOCAB = 256
EMBED_DIM = 128
BATCH = 16384

NC = 2
NS = 16
NW = NC * NS
B_PER_W = BATCH // NW
CHUNK = 64
N_CHUNKS = B_PER_W // CHUNK
NBUF = 6


def _build():
    mesh = plsc.VectorSubcoreMesh(
        core_axis_name="c", subcore_axis_name="s", num_cores=NC
    )

    @functools.partial(
        pl.kernel,
        mesh=mesh,
        out_type=jax.ShapeDtypeStruct((BATCH, EMBED_DIM), jnp.float32),
        scratch_types=[
            pltpu.VMEM((B_PER_W,), jnp.int32),
            pltpu.VMEM((NBUF, CHUNK, EMBED_DIM), jnp.float32),
            pltpu.VMEM_SHARED((VOCAB, EMBED_DIM), jnp.float32),
            pltpu.SemaphoreType.DMA((NBUF,)),
            pltpu.SemaphoreType.DMA((NBUF,)),
            pltpu.SemaphoreType.DMA,
            pltpu.SemaphoreType.DMA,
        ],
    )
    def gather_kernel(
        table_hbm, idx_hbm, out_hbm, idx_v, rows_v, table_sp, gsem, wsem,
        tsem, isem,
    ):
        cid = lax.axis_index("c")
        sid = lax.axis_index("s")
        wid = sid * NC + cid
        base = wid * B_PER_W

        rows_per_tile = VOCAB // NS
        trow = sid * rows_per_tile
        stage = pltpu.async_copy(
            table_hbm.at[pl.ds(trow, rows_per_tile)],
            table_sp.at[pl.ds(trow, rows_per_tile)],
            tsem,
        )
        iload = pltpu.async_copy(idx_hbm.at[pl.ds(base, B_PER_W)], idx_v, isem)
        stage.wait()
        iload.wait()
        plsc.subcore_barrier()

        gathers = [None] * N_CHUNKS
        writes = [None] * N_CHUNKS
        for j in range(N_CHUNKS + 1):
            if j < N_CHUNKS:
                s = j % NBUF
                if j >= NBUF:
                    writes[j - NBUF].wait()
                gathers[j] = pltpu.async_copy(
                    table_sp.at[idx_v.at[pl.ds(j * CHUNK, CHUNK)]],
                    rows_v.at[s],
                    gsem.at[s],
                )
            if j >= 1:
                jj = j - 1
                gathers[jj].wait()
                writes[jj] = pltpu.async_copy(
                    rows_v.at[jj % NBUF],
                    out_hbm.at[pl.ds(base + jj * CHUNK, CHUNK)],
                    wsem.at[jj % NBUF],
                )
        for w in writes[-NBUF:]:
            w.wait()

    return gather_kernel


@functools.cache
def _get_gather():
    return _build()


def kernel(input, table):
    idx = input.reshape(-1).astype(jnp.int32)
    return _get_gather()(table, idx)
